# baseline (speedup 1.0000x reference)
import functools

import jax
import jax.numpy as jnp
from jax import lax
from jax.experimental import pallas as pl
from jax.experimental.pallas import tpu as pltpu
from jax.experimental.pallas import tpu_sc as plsc

N = 1000
C = 81
M = 28
D = M * M
BM = 56
CT = C // 8


def _body(lab_ref, x_ref, o_ref):
    lab = lab_ref[...]
    labb = jnp.broadcast_to(lab, (8, N))
    sub = jax.lax.broadcasted_iota(jnp.int32, (8, N), 0)
    masks = [labb == (8 * t + sub) for t in range(CT)]
    last = lab == (C - 1)
    zero8 = jnp.zeros((8, N), jnp.float32)
    zero1 = jnp.zeros((1, N), jnp.float32)
    for m in range(BM):
        acc = zero8
        for t in range(CT):
            acc = acc + jnp.where(masks[t], x_ref[m, 8 * t : 8 * t + 8, :], zero8)
        row = jnp.sum(acc, axis=0, keepdims=True)
        row = row + jnp.where(last, x_ref[m, C - 1 : C, :], zero1)
        o_ref[pl.ds(m, 1), :] = jax.nn.sigmoid(row)


@jax.jit
def _select_sigmoid(lab2, xt):
    return pl.pallas_call(
        _body,
        grid=(D // BM,),
        in_specs=[
            pl.BlockSpec((1, N), lambda i: (0, 0)),
            pl.BlockSpec((BM, C, N), lambda i: (i, 0, 0)),
        ],
        out_specs=pl.BlockSpec((BM, N), lambda i: (i, 0)),
        out_shape=jax.ShapeDtypeStruct((D, N), jnp.float32),
    )(lab2, xt)


NW = 32
NLANE = 16
NP_N = 1024

_sc_mesh = plsc.VectorSubcoreMesh(core_axis_name="c", subcore_axis_name="s")


def _make_sc(ms_pad, m_base):
    mpw = ms_pad // NW

    @functools.partial(
        pl.kernel,
        mesh=_sc_mesh,
        out_type=jax.ShapeDtypeStruct((ms_pad, NP_N), jnp.float32),
        scratch_types=[
            pltpu.VMEM((NP_N,), jnp.int32),
            pltpu.VMEM((C, N), jnp.float32),
            pltpu.VMEM((NP_N,), jnp.float32),
            pltpu.SemaphoreType.DMA,
            pltpu.SemaphoreType.DMA,
        ],
    )
    def _sc_gather(lab_hbm, x_hbm, out_hbm, lab_v, buf, row, si, so):
        wid = lax.axis_index("s") * 2 + lax.axis_index("c")
        m0 = wid * mpw
        pltpu.sync_copy(lab_hbm, lab_v)

        def start_in(m):
            pltpu.async_copy(x_hbm.at[m], buf, si)

        def wait_in():
            pltpu.make_async_copy(x_hbm.at[0], buf, si).wait()

        start_in(jnp.minimum(m_base + m0, D - 1))

        lane = lax.iota(jnp.int32, NLANE)

        def do_group(base):
            acc = jnp.zeros((NLANE,), jnp.float32)
            labv = lab_v[pl.ds(base, NLANE)]
            for j in range(NLANE):
                cj = labv[j]
                vj = buf[cj, pl.ds(base, NLANE)]
                acc = jnp.where(lane == j, vj, acc)
            row[pl.ds(base, NLANE)] = 1.0 / (1.0 + jnp.exp(-acc))

        def grp(k, _):
            do_group(pl.multiple_of(k * NLANE, NLANE))
            return 0

        def body(i, _):
            m = m0 + i
            mn = jnp.minimum(m_base + m + 1, D - 1)
            wait_in()
            lax.fori_loop(0, (N - NLANE) // NLANE + 1, grp, 0)
            do_group(N - NLANE)
            pltpu.async_copy(row, out_hbm.at[m], so)
            start_in(mn)
            pltpu.make_async_copy(row, out_hbm.at[m], so).wait()
            return 0

        lax.fori_loop(0, mpw, body, 0)
        wait_in()

    return _sc_gather


_sc_full = _make_sc(800, 0)


def kernel(x, labels):
    xt = jnp.transpose(x, (2, 3, 1, 0)).reshape(D, C, N)
    lab_pad = jnp.pad(labels.astype(jnp.int32), (0, NP_N - N))
    out_t = _sc_full(lab_pad, xt)[:D, :N]
    return out_t.T.reshape(N, 1, M, M)


# speedup vs baseline: 1.9371x; 1.9371x over previous
import functools

import jax
import jax.numpy as jnp
from jax import lax
from jax.experimental import pallas as pl
from jax.experimental.pallas import tpu as pltpu
from jax.experimental.pallas import tpu_sc as plsc

N = 1000
C = 81
M = 28
D = M * M
BM = 48
CT = C // 8


def _body(lab_ref, x_ref, o_ref):
    lab = lab_ref[...]
    labb = jnp.broadcast_to(lab, (8, N))
    sub = jax.lax.broadcasted_iota(jnp.int32, (8, N), 0)
    masks = [labb == (8 * t + sub) for t in range(CT)]
    last = lab == (C - 1)
    zero8 = jnp.zeros((8, N), jnp.float32)
    zero1 = jnp.zeros((1, N), jnp.float32)
    for m in range(BM):
        acc = zero8
        for t in range(CT):
            acc = acc + jnp.where(masks[t], x_ref[m, 8 * t : 8 * t + 8, :], zero8)
        row = jnp.sum(acc, axis=0, keepdims=True)
        row = row + jnp.where(last, x_ref[m, C - 1 : C, :], zero1)
        o_ref[pl.ds(m, 1), :] = jax.nn.sigmoid(row)


def _select_sigmoid(lab2, xt, rows):
    return pl.pallas_call(
        _body,
        grid=(rows // BM,),
        in_specs=[
            pl.BlockSpec((1, N), lambda i: (0, 0)),
            pl.BlockSpec((BM, C, N), lambda i: (i, 0, 0)),
        ],
        out_specs=pl.BlockSpec((BM, N), lambda i: (i, 0)),
        out_shape=jax.ShapeDtypeStruct((rows, N), jnp.float32),
    )(lab2, xt)


NW = 32
NLANE = 16
NP_N = 1024

_sc_mesh = plsc.VectorSubcoreMesh(core_axis_name="c", subcore_axis_name="s")


def _make_sc(ms_pad, m_base):
    mpw = ms_pad // NW

    @functools.partial(
        pl.kernel,
        mesh=_sc_mesh,
        out_type=jax.ShapeDtypeStruct((ms_pad, NP_N), jnp.float32),
        scratch_types=[
            pltpu.VMEM((NP_N,), jnp.int32),
            pltpu.VMEM((C, N), jnp.float32),
            pltpu.VMEM((NP_N,), jnp.float32),
            pltpu.SemaphoreType.DMA,
            pltpu.SemaphoreType.DMA,
        ],
    )
    def _sc_gather(lab_hbm, x_hbm, out_hbm, lab_v, buf, row, si, so):
        wid = lax.axis_index("s") * 2 + lax.axis_index("c")
        m0 = wid * mpw
        pltpu.sync_copy(lab_hbm, lab_v)

        def start_in(m):
            pltpu.async_copy(x_hbm.at[m], buf, si)

        def wait_in():
            pltpu.make_async_copy(x_hbm.at[0], buf, si).wait()

        start_in(jnp.minimum(m_base + m0, D - 1))

        lane = lax.iota(jnp.int32, NLANE)

        def do_group(base):
            acc = jnp.zeros((NLANE,), jnp.float32)
            labv = lab_v[pl.ds(base, NLANE)]
            for j in range(NLANE):
                cj = labv[j]
                vj = buf[cj, pl.ds(base, NLANE)]
                acc = jnp.where(lane == j, vj, acc)
            row[pl.ds(base, NLANE)] = 1.0 / (1.0 + jnp.exp(-acc))

        def grp(k, _):
            do_group(pl.multiple_of(k * NLANE, NLANE))
            return 0

        n_grp = (N - NLANE) // NLANE + 1

        def body(i, _):
            m = m0 + i
            mn = jnp.minimum(m_base + m + 1, D - 1)
            wait_in()
            lax.fori_loop(0, n_grp, grp, 0, unroll=2)
            do_group(N - NLANE)
            pltpu.async_copy(row, out_hbm.at[m], so)
            start_in(mn)
            pltpu.make_async_copy(row, out_hbm.at[m], so).wait()
            return 0

        lax.fori_loop(0, mpw, body, 0)
        wait_in()

    return _sc_gather


MS = 256
MT = D - MS

_sc_part = _make_sc(MS, MT)


def kernel(x, labels):
    xt = jnp.transpose(x, (2, 3, 1, 0)).reshape(D, C, N)
    lab32 = labels.astype(jnp.int32)
    lab_pad = jnp.pad(lab32, (0, NP_N - N))
    out_sc = _sc_part(lab_pad, xt)[:, :N]
    out_tc = _select_sigmoid(lab32.reshape(1, N), xt, MT)
    out_t = jnp.concatenate([out_tc, out_sc], axis=0)
    return out_t.T.reshape(N, 1, M, M)


# speedup vs baseline: 2.2900x; 1.1822x over previous
import jax
import jax.numpy as jnp
from jax.experimental import pallas as pl

N = 1000
C = 81
M = 28
D = M * M
BM = 56
CT = C // 8


def _body(lab_ref, x_ref, o_ref):
    lab = lab_ref[...]
    labb = jnp.broadcast_to(lab, (8, N))
    sub = jax.lax.broadcasted_iota(jnp.int32, (8, N), 0)
    masks = [labb == (8 * t + sub) for t in range(CT)]
    last = lab == (C - 1)
    zero8 = jnp.zeros((8, N), jnp.float32)
    zero1 = jnp.zeros((1, N), jnp.float32)
    for m in range(BM):
        acc = zero8
        for t in range(CT):
            acc = acc + jnp.where(masks[t], x_ref[m, 8 * t : 8 * t + 8, :], zero8)
        row = jnp.sum(acc, axis=0, keepdims=True)
        row = row + jnp.where(last, x_ref[m, C - 1 : C, :], zero1)
        o_ref[pl.ds(m, 1), :] = jax.nn.sigmoid(row)


def _select_sigmoid(lab2, xt):
    return pl.pallas_call(
        _body,
        grid=(D // BM,),
        in_specs=[
            pl.BlockSpec((1, N), lambda i: (0, 0)),
            pl.BlockSpec((BM, C, N), lambda i: (i, 0, 0)),
        ],
        out_specs=pl.BlockSpec((BM, N), lambda i: (i, 0)),
        out_shape=jax.ShapeDtypeStruct((D, N), jnp.float32),
    )(lab2, xt)


def kernel(x, labels):
    xt = jnp.transpose(x, (2, 3, 1, 0)).reshape(D, C, N)
    lab2 = labels.astype(jnp.int32).reshape(1, N)
    out_t = _select_sigmoid(lab2, xt)
    return out_t.T.reshape(N, 1, M, M)
